# baseline (speedup 1.0000x reference)
import functools

import jax
import jax.numpy as jnp
from jax import lax
from jax.experimental import pallas as pl
from jax.experimental.pallas import tpu as pltpu
from jax.experimental.pallas import tpu_sc as plsc

_VOCAB = 1000000
_EMBED = 64
_SEQ = 200
_BATCH = 4096
_LANES = 16
_EV = _EMBED // _LANES

_NC, _NS = 2, 16
_NW = _NC * _NS
_ROWS = _BATCH * _SEQ
_R_PER_W = _ROWS // _NW
_SEQ_PER_CHUNK = 2
_CH = _SEQ_PER_CHUNK * _SEQ
_NCHUNK = _R_PER_W // _CH
_G = 5
_GSZ = _CH // _G


def _emb_body(x_hbm, table_hbm, pe_hbm, out_hbm, idx_v, rows_v, pe_v, sem):
    wid = lax.axis_index("s") * _NC + lax.axis_index("c")
    base_w = wid * _R_PER_W

    pltpu.sync_copy(pe_hbm, pe_v)

    def chunk_body(g, carry):
        base = base_w + g * _CH
        pltpu.sync_copy(x_hbm.at[pl.ds(base, _CH)], idx_v)
        cps = [
            pltpu.async_copy(
                table_hbm.at[idx_v.at[pl.ds(j * _GSZ, _GSZ)]],
                rows_v.at[pl.ds(j * _GSZ, _GSZ)],
                sem,
            )
            for j in range(_G)
        ]
        for cp in cps:
            cp.wait()

        def add_body(i, c2):
            for s in range(_SEQ_PER_CHUNK):
                for c in range(_EV):
                    sl = pl.ds(c * _LANES, _LANES)
                    rows_v[s * _SEQ + i, sl] = rows_v[s * _SEQ + i, sl] + pe_v[i, sl]
            return c2

        lax.fori_loop(0, _SEQ, add_body, 0)
        pltpu.sync_copy(rows_v, out_hbm.at[pl.ds(base, _CH)])
        return carry

    lax.fori_loop(0, _NCHUNK, chunk_body, 0)


_emb = functools.partial(
    pl.kernel,
    out_type=jax.ShapeDtypeStruct((_ROWS, _EMBED), jnp.float32),
    mesh=plsc.VectorSubcoreMesh(core_axis_name="c", subcore_axis_name="s"),
    compiler_params=pltpu.CompilerParams(use_tc_tiling_on_sc=False),
    scratch_types=[
        pltpu.VMEM((_CH,), jnp.int32),
        pltpu.VMEM((_CH, _EMBED), jnp.float32),
        pltpu.VMEM((_SEQ, _EMBED), jnp.float32),
        pltpu.SemaphoreType.DMA,
    ],
)(_emb_body)


def kernel(x, table, pe):
    xf = x.reshape(_ROWS)
    out = _emb(xf, table, pe)
    return out.reshape(_BATCH, _SEQ, _EMBED)


# speedup vs baseline: 1.1246x; 1.1246x over previous
import functools

import jax
import jax.numpy as jnp
from jax import lax
from jax.experimental import pallas as pl
from jax.experimental.pallas import tpu as pltpu
from jax.experimental.pallas import tpu_sc as plsc

_VOCAB = 1000000
_EMBED = 64
_SEQ = 200
_BATCH = 4096
_LANES = 16
_EV = _EMBED // _LANES

_NC, _NS = 2, 16
_NW = _NC * _NS
_ROWS = _BATCH * _SEQ
_R_PER_W = _ROWS // _NW
_SEQ_PER_CHUNK = 2
_CH = _SEQ_PER_CHUNK * _SEQ
_NCHUNK = _R_PER_W // _CH
_PLAN = ((0, 128), (128, 128), (256, 128), (384, 16))


def _emb_body(x_hbm, table_hbm, pe_hbm, out_hbm,
              idx_v, rows0, rows1, pe_v, semg0, semg1, semsc0, semsc1):
    wid = lax.axis_index("s") * _NC + lax.axis_index("c")
    base_w = wid * _R_PER_W
    rows = (rows0, rows1)
    semg = (semg0, semg1)
    semsc = (semsc0, semsc1)

    pltpu.sync_copy(pe_hbm, pe_v)
    pltpu.sync_copy(x_hbm.at[pl.ds(base_w, _R_PER_W)], idx_v)

    def fire_gather(g, b):
        for off, sz in _PLAN:
            pltpu.async_copy(
                table_hbm.at[idx_v.at[pl.ds(g * _CH + off, sz)]],
                rows[b].at[pl.ds(off, sz)],
                semg[b],
            )

    def drain_gather(b):
        pltpu.make_async_copy(out_hbm.at[pl.ds(0, _CH)], rows[b], semg[b]).wait()

    def drain_scatter(b):
        pltpu.make_async_copy(rows[b], out_hbm.at[pl.ds(0, _CH)], semsc[b]).wait()

    fire_gather(0, 0)

    @pl.loop(0, _NCHUNK, step=2)
    def _chunks(g0):
        for b in range(2):
            g = g0 + b
            drain_gather(b)

            @pl.when(g + 1 < _NCHUNK)
            def _():
                @pl.when(g >= 1)
                def _():
                    drain_scatter(1 - b)

                fire_gather(g + 1, 1 - b)

            rb = rows[b]

            @plsc.parallel_loop(0, _SEQ, unroll=2)
            def _add(i):
                for c in range(_EV):
                    sl = pl.ds(c * _LANES, _LANES)
                    p = pe_v[i, sl]
                    for s in range(_SEQ_PER_CHUNK):
                        rb[s * _SEQ + i, sl] = rb[s * _SEQ + i, sl] + p

            pltpu.async_copy(rb, out_hbm.at[pl.ds(base_w + g * _CH, _CH)], semsc[b])

    drain_scatter(0)
    drain_scatter(1)


_emb = functools.partial(
    pl.kernel,
    out_type=jax.ShapeDtypeStruct((_ROWS, _EMBED), jnp.float32),
    mesh=plsc.VectorSubcoreMesh(core_axis_name="c", subcore_axis_name="s"),
    compiler_params=pltpu.CompilerParams(use_tc_tiling_on_sc=False),
    scratch_types=[
        pltpu.VMEM((_R_PER_W,), jnp.int32),
        pltpu.VMEM((_CH, _EMBED), jnp.float32),
        pltpu.VMEM((_CH, _EMBED), jnp.float32),
        pltpu.VMEM((_SEQ, _EMBED), jnp.float32),
        pltpu.SemaphoreType.DMA,
        pltpu.SemaphoreType.DMA,
        pltpu.SemaphoreType.DMA,
        pltpu.SemaphoreType.DMA,
    ],
)(_emb_body)


def kernel(x, table, pe):
    xf = x.reshape(_ROWS)
    out = _emb(xf, table, pe)
    return out.reshape(_BATCH, _SEQ, _EMBED)


# speedup vs baseline: 1.1387x; 1.0126x over previous
import functools

import jax
import jax.numpy as jnp
from jax import lax
from jax.experimental import pallas as pl
from jax.experimental.pallas import tpu as pltpu
from jax.experimental.pallas import tpu_sc as plsc

_VOCAB = 1000000
_EMBED = 64
_SEQ = 200
_BATCH = 4096
_LANES = 16

_NC, _NS = 2, 16
_NW = _NC * _NS
_BB = _BATCH // _NW
_GROUPS = _BB // _LANES
_DT = _EMBED // 8


def _emb_body(xtf_hbm, table_hbm, pe_hbm, out_hbm,
              idx0, idx1, g0, g1, o0, o1, pe_v,
              semi0, semi1, semg0, semg1, semo0, semo1):
    w = lax.axis_index("s") * _NC + lax.axis_index("c")
    col0 = w * _BB
    idx = (idx0, idx1)
    G = (g0, g1)
    O = (o0, o1)
    semi = (semi0, semi1)
    semg = (semg0, semg1)
    semo = (semo0, semo1)

    pltpu.sync_copy(pe_hbm, pe_v)

    iota = lax.iota(jnp.int32, _LANES)

    def fire_idx(t, b):
        pltpu.async_copy(
            xtf_hbm.at[pl.ds(t * _BATCH + col0, _BB)], idx[b], semi[b])

    def wait_idx(b):
        pltpu.make_async_copy(xtf_hbm.at[pl.ds(0, _BB)], idx[b], semi[b]).wait()

    def fire_gather(b):
        pltpu.async_copy(table_hbm.at[idx[b]], G[b], semg[b])

    def wait_gather(b):
        pltpu.make_async_copy(table_hbm.at[pl.ds(0, _BB)], G[b], semg[b]).wait()

    def fire_out(t, b):
        pltpu.async_copy(O[b], out_hbm.at[t, :, w], semo[b])

    def wait_out(b):
        pltpu.make_async_copy(O[b], out_hbm.at[0, :, 0], semo[b]).wait()

    fire_idx(0, 0)
    fire_idx(1, 1)
    wait_idx(0)
    fire_gather(0)

    @pl.loop(0, _SEQ, step=2)
    def _steps(t0):
        for b in range(2):
            t = t0 + b
            nb = 1 - b
            wait_gather(b)

            @pl.when(t + 1 < _SEQ)
            def _():
                wait_idx(nb)
                fire_gather(nb)

            @pl.when(t + 2 < _SEQ)
            def _():
                fire_idx(t + 2, b)

            @pl.when(t >= 2)
            def _():
                wait_out(b)

            @plsc.parallel_loop(0, _EMBED, unroll=2)
            def _transpose(d):
                p = plsc.load_gather(pe_v, [jnp.broadcast_to(t * _EMBED + d, (_LANES,))])
                cols = jnp.broadcast_to(d, (_LANES,))
                for g in range(_GROUPS):
                    v = plsc.load_gather(G[b], [iota + g * _LANES, cols])
                    O[b][d // 8, d % 8, pl.ds(g * _LANES, _LANES)] = v + p

            fire_out(t, b)

    wait_out(0)
    wait_out(1)


_emb = functools.partial(
    pl.kernel,
    out_type=jax.ShapeDtypeStruct((_SEQ, _DT, _NW, 8, _BB), jnp.float32),
    mesh=plsc.VectorSubcoreMesh(core_axis_name="c", subcore_axis_name="s"),
    compiler_params=pltpu.CompilerParams(
        use_tc_tiling_on_sc=False, needs_layout_passes=False),
    scratch_types=[
        pltpu.VMEM((_BB,), jnp.int32),
        pltpu.VMEM((_BB,), jnp.int32),
        pltpu.VMEM((_BB, 128), jnp.float32),
        pltpu.VMEM((_BB, 128), jnp.float32),
        pltpu.VMEM((_DT, 8, _BB), jnp.float32),
        pltpu.VMEM((_DT, 8, _BB), jnp.float32),
        pltpu.VMEM((_SEQ * _EMBED,), jnp.float32),
        pltpu.SemaphoreType.DMA,
        pltpu.SemaphoreType.DMA,
        pltpu.SemaphoreType.DMA,
        pltpu.SemaphoreType.DMA,
        pltpu.SemaphoreType.DMA,
        pltpu.SemaphoreType.DMA,
    ],
)(_emb_body)


def kernel(x, table, pe):
    xtf = jnp.transpose(x).reshape(_SEQ * _BATCH)
    table2 = jnp.pad(table, ((0, 0), (0, 128 - _EMBED)))
    pef = pe.reshape(_SEQ * _EMBED)
    out5 = _emb(xtf, table2, pef)
    return out5.transpose((2, 4, 0, 1, 3)).reshape(_BATCH, _SEQ, _EMBED)
